# baseline (speedup 1.0000x reference)
import functools

import jax
import jax.numpy as jnp
import numpy as np
from jax import lax
from jax.experimental import pallas as pl
from jax.experimental.pallas import tpu as pltpu
from jax.experimental.pallas import tpu_sc as plsc

G = 8
L = 4
R = 4
H = 64
W = 128
C = 64
COLS = 256
LEV_OFF = (0, 128, 192, 224)
LEV_D = (128, 64, 32, 16)
SLAB = G * 16 * COLS
OSLAB = G * 16 * 2 * L * 9
N_UNITS = H * (W // 16)
INV_SQRT_G = 1.0 / np.sqrt(float(G))


def _avg_matrix():
    P = np.zeros((128, 240), dtype=np.float32)
    col = 0
    for i in range(4):
        d = 128 >> i
        s = 1 << i
        for e in range(d):
            P[e * s:(e + 1) * s, col + e] = 1.0 / s
        col += d
    return P


def _table_matrices():
    P = _avg_matrix()
    Paug = np.concatenate([P, np.zeros((128, 16), np.float32)], axis=1)
    pf = Paug * INV_SQRT_G
    q = np.zeros((3, 128, 256), np.float32)
    for kd in range(3):
        Sh = np.zeros((128, 128), np.float32)
        for w2 in range(128):
            s = w2 + kd - 1
            if 0 <= s < 128:
                Sh[s, w2] = 1.0
        q[kd] = Sh @ Paug
    return jnp.asarray(pf), jnp.asarray(q)


def _tc_tables_body(f1m, f1c, f1p, f2m, f2c, f2p, feat0_ref, w72_ref,
                    proj_ref, pf_ref, q_ref, ftab_ref, gtab_ref, *, h0):
    h = pl.program_id(0) + h0
    cvs = []
    for ky, (fa, fb) in enumerate(((f1m, f2m), (f1c, f2c), (f1p, f2p))):
        hh = h + ky - 1
        a = fa[...].reshape(G, 8, W).astype(jnp.bfloat16)
        b = fb[...].reshape(G, 8, W).astype(jnp.bfloat16)
        if ky != 1:
            scale = jnp.where((hh >= 0) & (hh < H), 1.0, 0.0).astype(jnp.bfloat16)
            a = a * scale
        row = []
        for g in range(G):
            cvg = lax.dot_general(a[g], b[g], (((0,), (0,)), ((), ())),
                                  preferred_element_type=jnp.float32)
            row.append(cvg)
        cvs.append(row)

    zr = jnp.zeros((1, 128), jnp.float32)
    slabs = []
    for ky in range(3):
        for g in range(G):
            s = cvs[ky][g]
            slabs.append(jnp.concatenate([zr, s[:-1]], axis=0))
            slabs.append(s)
            slabs.append(jnp.concatenate([s[1:], zr], axis=0))
    X72 = jnp.stack(slabs).astype(jnp.bfloat16)

    Z = lax.dot_general(w72_ref[...], X72, (((0,), (0,)), ((), ())),
                        preferred_element_type=jnp.float32)

    pbT = lax.dot_general(feat0_ref[...], proj_ref[...],
                          (((0,), (1,)), ((), ())),
                          preferred_element_type=jnp.float32)

    for g in range(G):
        Fall = lax.dot_general(cvs[1][g].astype(jnp.bfloat16), pf_ref[...],
                               (((1,), (0,)), ((), ())),
                               preferred_element_type=jnp.float32)
        ftab_ref[0, :, g] = Fall.reshape(8, 16, COLS)
        acc = jnp.broadcast_to(pbT[:, g:g + 1], (128, COLS))
        for kd in range(3):
            acc = acc + lax.dot_general(Z[g * 3 + kd], q_ref[kd],
                                        (((1,), (0,)), ((), ())),
                                        preferred_element_type=jnp.float32)
        gtab_ref[0, :, g] = acc.reshape(8, 16, COLS)


def _build_tables(f1, f2, feat0, w72, proj_w, pf, q, h0, nh):
    full = lambda s: pl.BlockSpec(s, lambda h: (0,) * len(s))
    hm = pl.BlockSpec((C, W), lambda h: (0, jnp.maximum(h0 + h - 1, 0)))
    hc = pl.BlockSpec((C, W), lambda h: (0, h0 + h))
    hp = pl.BlockSpec((C, W), lambda h: (0, jnp.minimum(h0 + h + 1, H - 1)))
    return pl.pallas_call(
        functools.partial(_tc_tables_body, h0=h0),
        grid=(nh,),
        in_specs=[
            hm, hc, hp,
            hm, hc, hp,
            hc,
            full((72, 24)),
            full((G, C)),
            full((128, COLS)),
            full((3, 128, COLS)),
        ],
        out_specs=[
            pl.BlockSpec((1, 8, G, 16, COLS), lambda h: (h, 0, 0, 0, 0)),
            pl.BlockSpec((1, 8, G, 16, COLS), lambda h: (h, 0, 0, 0, 0)),
        ],
        out_shape=[
            jax.ShapeDtypeStruct((nh, 8, G, 16, COLS), jnp.float32),
            jax.ShapeDtypeStruct((nh, 8, G, 16, COLS), jnp.float32),
        ],
        compiler_params=pltpu.CompilerParams(
            dimension_semantics=("arbitrary",),
        ),
    )(f1, f1, f1, f2, f2, f2, feat0, w72, proj_w, pf, q)


def _flat2(x):
    return x.reshape(C, H * W)


def _sc_sample_body(ftab, gtab, coords_r, out_hbm,
                    abuf, bbuf, obuf, cbuf, i0buf, wbuf, sema, semb,
                    *, n_units):
    nc = 2
    wid = lax.axis_index("s") * nc + lax.axis_index("c")
    units_per = n_units // 32
    lane = lax.iota(jnp.int32, 16)

    pltpu.sync_copy(coords_r.at[pl.ds(wid * units_per * 16, units_per * 16)],
                    cbuf)
    u0 = wid * units_per
    pltpu.async_copy(ftab.at[u0 // 8, u0 % 8], abuf, sema)

    def sample_pass(src_ref, pass_t):
        def g_body(g, _):
            gsplat = jnp.full((16,), 0, jnp.int32) + g
            for i in range(L):
                wf = wbuf[pl.ds(i * 16, 16)]
                win = []
                for jw in range(10):
                    idx = i0buf[pl.ds((i * 10 + jw) * 16, 16)]
                    win.append(plsc.load_gather(src_ref, [gsplat, lane, idx]))
                for k in range(9):
                    col0 = (2 * i + pass_t) * 9 + k
                    v = win[k] + (win[k + 1] - win[k]) * wf
                    obuf[pl.ds(col0 * 128 + g * 16, 16)] = v
            return 0

        lax.fori_loop(0, G, g_body, 0)

    def unit_body(j, _):
        u = wid * units_per + j
        c = cbuf[pl.ds(j * 16, 16)]
        for i in range(L):
            ci = c * (0.5 ** i)
            di = LEV_D[i]
            off = LEV_OFF[i]
            t0 = ci.astype(jnp.int32)
            fl = jnp.where(t0.astype(jnp.float32) > ci, t0 - 1, t0)
            wbuf[pl.ds(i * 16, 16)] = ci - fl.astype(jnp.float32)
            for jw in range(10):
                i0buf[pl.ds((i * 10 + jw) * 16, 16)] = (
                    jnp.clip(fl + (jw - R), 0, di - 1) + off)

        pltpu.make_async_copy(ftab.at[u // 8, u % 8], abuf, sema).wait()
        pltpu.async_copy(gtab.at[u // 8, u % 8], bbuf, semb)
        sample_pass(abuf, 0)
        pltpu.make_async_copy(gtab.at[u // 8, u % 8], bbuf, semb).wait()

        @pl.when(j < units_per - 1)
        def _():
            pltpu.async_copy(ftab.at[(u + 1) // 8, (u + 1) % 8], abuf, sema)

        sample_pass(bbuf, 1)
        pltpu.sync_copy(obuf, out_hbm.at[pl.ds(u * OSLAB, OSLAB)])
        return 0

    lax.fori_loop(0, units_per, unit_body, 0)


def _sc_sample(ftab, gtab, coords_flat, n_units):
    mesh = plsc.VectorSubcoreMesh(core_axis_name="c", subcore_axis_name="s")
    kern = functools.partial(
        pl.kernel,
        out_type=jax.ShapeDtypeStruct((n_units * OSLAB,), jnp.float32),
        mesh=mesh,
        scratch_types=[
            pltpu.VMEM((G, 16, COLS), jnp.float32),
            pltpu.VMEM((G, 16, COLS), jnp.float32),
            pltpu.VMEM((OSLAB,), jnp.float32),
            pltpu.VMEM((n_units // 32 * 16,), jnp.float32),
            pltpu.VMEM((L * 10 * 16,), jnp.int32),
            pltpu.VMEM((L * 16,), jnp.float32),
            pltpu.SemaphoreType.DMA,
            pltpu.SemaphoreType.DMA,
        ],
        compiler_params=pltpu.CompilerParams(needs_layout_passes=False),
    )(functools.partial(_sc_sample_body, n_units=n_units))
    return kern(ftab, gtab, coords_flat)


def kernel(fmap1, fmap2, feat0, conv_w, proj_w, coords):
    w72 = (conv_w.transpose(3, 1, 4, 0, 2).reshape(72, 24)
           * INV_SQRT_G).astype(jnp.bfloat16)
    pf, q = _table_matrices()
    pf = pf.astype(jnp.bfloat16)
    coords_r = coords[0, 0].reshape(-1)

    f1, f2, ft0 = _flat2(fmap1[0]), _flat2(fmap2[0]), _flat2(feat0[0])
    npc = 4
    nh = H // npc
    nu = nh * 8
    tabs = [_build_tables(f1, f2, ft0, w72, proj_w, pf, q, p * nh, nh)
            for p in range(npc)]
    outs = [_sc_sample(ft, gt, coords_r[p * nh * W:(p + 1) * nh * W], nu)
            for p, (ft, gt) in enumerate(tabs)]

    def _piece(o):
        return (o.reshape(nh, 8, L, 2, 9, G, 16)
                .transpose(2, 3, 5, 4, 0, 1, 6)
                .reshape(2 * L * G * 9, nh, W))

    out = jnp.concatenate([_piece(o) for o in outs], axis=1)
    return out[None]


# speedup vs baseline: 1.4247x; 1.4247x over previous
import functools

import jax
import jax.numpy as jnp
import numpy as np
from jax import lax
from jax.experimental import pallas as pl
from jax.experimental.pallas import tpu as pltpu
from jax.experimental.pallas import tpu_sc as plsc

G = 8
L = 4
R = 4
H = 64
W = 128
C = 64
COLS = 256
LEV_OFF = (0, 128, 192, 224)
LEV_D = (128, 64, 32, 16)
SLAB = G * 16 * COLS
OSLAB = G * 16 * 2 * L * 9
N_UNITS = H * (W // 16)
INV_SQRT_G = 1.0 / np.sqrt(float(G))


def _avg_matrix():
    P = np.zeros((128, 240), dtype=np.float32)
    col = 0
    for i in range(4):
        d = 128 >> i
        s = 1 << i
        for e in range(d):
            P[e * s:(e + 1) * s, col + e] = 1.0 / s
        col += d
    return P


def _table_matrices():
    P = _avg_matrix()
    Paug = np.concatenate([P, np.zeros((128, 16), np.float32)], axis=1)
    pf = Paug * INV_SQRT_G
    q = np.zeros((3, 128, 256), np.float32)
    for kd in range(3):
        Sh = np.zeros((128, 128), np.float32)
        for w2 in range(128):
            s = w2 + kd - 1
            if 0 <= s < 128:
                Sh[s, w2] = 1.0
        q[kd] = Sh @ Paug
    return jnp.asarray(pf), jnp.asarray(q)


def _tc_tables_body(f1m, f1c, f1p, f2m, f2c, f2p, feat0_ref, w72_ref,
                    proj_ref, pf_ref, q_ref, ftab_ref, gtab_ref, *, h0):
    h = pl.program_id(0) + h0
    cvs = []
    for ky, (fa, fb) in enumerate(((f1m, f2m), (f1c, f2c), (f1p, f2p))):
        hh = h + ky - 1
        a = fa[...].reshape(G, 8, W).astype(jnp.bfloat16)
        b = fb[...].reshape(G, 8, W).astype(jnp.bfloat16)
        if ky != 1:
            scale = jnp.where((hh >= 0) & (hh < H), 1.0, 0.0).astype(jnp.bfloat16)
            a = a * scale
        row = []
        for g in range(G):
            cvg = lax.dot_general(a[g], b[g], (((0,), (0,)), ((), ())),
                                  preferred_element_type=jnp.float32)
            row.append(cvg)
        cvs.append(row)

    zr = jnp.zeros((1, 128), jnp.float32)
    slabs = []
    for ky in range(3):
        for g in range(G):
            s = cvs[ky][g]
            slabs.append(jnp.concatenate([zr, s[:-1]], axis=0))
            slabs.append(s)
            slabs.append(jnp.concatenate([s[1:], zr], axis=0))
    X72 = jnp.stack(slabs).astype(jnp.bfloat16)

    Z = lax.dot_general(w72_ref[...], X72, (((0,), (0,)), ((), ())),
                        preferred_element_type=jnp.float32)

    pbT = lax.dot_general(feat0_ref[...], proj_ref[...],
                          (((0,), (1,)), ((), ())),
                          preferred_element_type=jnp.float32)

    for g in range(G):
        Fall = lax.dot_general(cvs[1][g].astype(jnp.bfloat16), pf_ref[...],
                               (((1,), (0,)), ((), ())),
                               preferred_element_type=jnp.float32)
        ftab_ref[0, :, g] = Fall.reshape(8, 16, COLS)
        acc = jnp.broadcast_to(pbT[:, g:g + 1], (128, COLS))
        for kd in range(3):
            acc = acc + lax.dot_general(Z[g * 3 + kd], q_ref[kd],
                                        (((1,), (0,)), ((), ())),
                                        preferred_element_type=jnp.float32)
        gtab_ref[0, :, g] = acc.reshape(8, 16, COLS)


def _build_tables(f1, f2, feat0, w72, proj_w, pf, q, h0, nh):
    full = lambda s: pl.BlockSpec(s, lambda h: (0,) * len(s))
    hm = pl.BlockSpec((C, W), lambda h: (0, jnp.maximum(h0 + h - 1, 0)))
    hc = pl.BlockSpec((C, W), lambda h: (0, h0 + h))
    hp = pl.BlockSpec((C, W), lambda h: (0, jnp.minimum(h0 + h + 1, H - 1)))
    return pl.pallas_call(
        functools.partial(_tc_tables_body, h0=h0),
        grid=(nh,),
        in_specs=[
            hm, hc, hp,
            hm, hc, hp,
            hc,
            full((72, 24)),
            full((G, C)),
            full((128, COLS)),
            full((3, 128, COLS)),
        ],
        out_specs=[
            pl.BlockSpec((1, 8, G, 16, COLS), lambda h: (h, 0, 0, 0, 0)),
            pl.BlockSpec((1, 8, G, 16, COLS), lambda h: (h, 0, 0, 0, 0)),
        ],
        out_shape=[
            jax.ShapeDtypeStruct((nh, 8, G, 16, COLS), jnp.float32),
            jax.ShapeDtypeStruct((nh, 8, G, 16, COLS), jnp.float32),
        ],
        compiler_params=pltpu.CompilerParams(
            dimension_semantics=("arbitrary",),
        ),
    )(f1, f1, f1, f2, f2, f2, feat0, w72, proj_w, pf, q)


def _flat2(x):
    return x.reshape(C, H * W)


def _sc_sample_body(ftab, gtab, coords_r, out_hbm,
                    abuf, bbuf, obuf, cbuf, i0buf, wbuf, sema, semb,
                    *, n_units):
    nc = 2
    wid = lax.axis_index("s") * nc + lax.axis_index("c")
    units_per = n_units // 32
    lane = lax.iota(jnp.int32, 16)

    pltpu.sync_copy(coords_r.at[pl.ds(wid * units_per * 16, units_per * 16)],
                    cbuf)
    u0 = wid * units_per
    pltpu.async_copy(ftab.at[u0 // 8, u0 % 8], abuf, sema)

    def sample_pass(src_ref, pass_t):
        def g_body(g, _):
            gsplat = jnp.full((16,), 0, jnp.int32) + g
            for i in range(L):
                wf = wbuf[pl.ds(i * 16, 16)]
                win = []
                for jw in range(10):
                    idx = i0buf[pl.ds((i * 10 + jw) * 16, 16)]
                    win.append(plsc.load_gather(src_ref, [gsplat, lane, idx]))
                for k in range(9):
                    col0 = (2 * i + pass_t) * 9 + k
                    v = win[k] + (win[k + 1] - win[k]) * wf
                    obuf[pl.ds(col0 * 128 + g * 16, 16)] = v
            return 0

        lax.fori_loop(0, G, g_body, 0)

    def unit_body(j, _):
        u = wid * units_per + j
        c = cbuf[pl.ds(j * 16, 16)]
        for i in range(L):
            ci = c * (0.5 ** i)
            di = LEV_D[i]
            off = LEV_OFF[i]
            t0 = ci.astype(jnp.int32)
            fl = jnp.where(t0.astype(jnp.float32) > ci, t0 - 1, t0)
            wbuf[pl.ds(i * 16, 16)] = ci - fl.astype(jnp.float32)
            for jw in range(10):
                i0buf[pl.ds((i * 10 + jw) * 16, 16)] = (
                    jnp.clip(fl + (jw - R), 0, di - 1) + off)

        pltpu.make_async_copy(ftab.at[u // 8, u % 8], abuf, sema).wait()
        pltpu.async_copy(gtab.at[u // 8, u % 8], bbuf, semb)
        sample_pass(abuf, 0)
        pltpu.make_async_copy(gtab.at[u // 8, u % 8], bbuf, semb).wait()

        @pl.when(j < units_per - 1)
        def _():
            pltpu.async_copy(ftab.at[(u + 1) // 8, (u + 1) % 8], abuf, sema)

        sample_pass(bbuf, 1)
        pltpu.sync_copy(obuf, out_hbm.at[pl.ds(u * OSLAB, OSLAB)])
        return 0

    lax.fori_loop(0, units_per, unit_body, 0)


def _sc_sample(ftab, gtab, coords_flat, n_units):
    mesh = plsc.VectorSubcoreMesh(core_axis_name="c", subcore_axis_name="s")
    kern = functools.partial(
        pl.kernel,
        out_type=jax.ShapeDtypeStruct((n_units * OSLAB,), jnp.float32),
        mesh=mesh,
        scratch_types=[
            pltpu.VMEM((G, 16, COLS), jnp.float32),
            pltpu.VMEM((G, 16, COLS), jnp.float32),
            pltpu.VMEM((OSLAB,), jnp.float32),
            pltpu.VMEM((n_units // 32 * 16,), jnp.float32),
            pltpu.VMEM((L * 10 * 16,), jnp.int32),
            pltpu.VMEM((L * 16,), jnp.float32),
            pltpu.SemaphoreType.DMA,
            pltpu.SemaphoreType.DMA,
        ],
        compiler_params=pltpu.CompilerParams(needs_layout_passes=False),
    )(functools.partial(_sc_sample_body, n_units=n_units))
    return kern(ftab, gtab, coords_flat)


def kernel(fmap1, fmap2, feat0, conv_w, proj_w, coords):
    w72 = (conv_w.transpose(3, 1, 4, 0, 2).reshape(72, 24)
           * INV_SQRT_G).astype(jnp.bfloat16)
    pf, q = _table_matrices()
    pf = pf.astype(jnp.bfloat16)
    coords_r = coords[0, 0].reshape(-1)

    f1, f2, ft0 = _flat2(fmap1[0]), _flat2(fmap2[0]), _flat2(feat0[0])
    nh = H // 2
    nu = nh * 8
    fa, ga = _build_tables(f1, f2, ft0, w72, proj_w, pf, q, 0, nh)
    fb, gb = _build_tables(f1, f2, ft0, w72, proj_w, pf, q, nh, nh)
    oa = _sc_sample(fa, ga, coords_r[:nh * W], nu)
    ob = _sc_sample(fb, gb, coords_r[nh * W:], nu)

    def _half(o):
        return (o.reshape(nh, 8, L, 2, 9, G, 16)
                .transpose(2, 3, 5, 4, 0, 1, 6)
                .reshape(2 * L * G * 9, nh, W))

    out = jnp.concatenate([_half(oa), _half(ob)], axis=1)
    return out[None]
